# Initial kernel scaffold; baseline (speedup 1.0000x reference)
import jax
import jax.numpy as jnp
from jax.experimental import pallas as pl


def kernel(x, edge_index, W_init, W1, b1, Wres1, bres1, W2, b2, Wres2, bres2, batch_size):
    raise NotImplementedError("write your pallas kernel here")


# speedup vs baseline: 2.7749x; 2.7749x over previous
import functools

import jax
import jax.numpy as jnp
from jax import lax
from jax.experimental import pallas as pl
from jax.experimental.pallas import tpu as pltpu
from jax.experimental.pallas import tpu_sc as plsc

N_NODES = 10000
N_EDGES = 160000
LANE = 128
NS = 16
NC = 2
NB = 80
EPS = NB * LANE
EPAD = EPS * NS
N_PAD = 10240
ZROWS = N_PAD // NS
MB = 1000


def _seg_sum_body(npc, hflat, srcall, dst3, zeros, out, src_v, dst_v, rows_v,
                  agg, sem):
    c = lax.axis_index("c")
    s = lax.axis_index("s")
    pltpu.sync_copy(dst3.at[s], dst_v)
    for j in range(npc):
        chunk = c * npc + j
        pltpu.sync_copy(zeros, agg.at[pl.ds(s * ZROWS, ZROWS)])
        pltpu.sync_copy(srcall.at[chunk * NS + s], src_v)
        plsc.subcore_barrier()

        def body(b, carry):
            pltpu.async_copy(hflat.at[src_v.at[b]], rows_v, sem).wait()
            pltpu.sync_copy(rows_v, agg.at[dst_v.at[b]], add=True)
            return carry

        lax.fori_loop(0, NB, body, 0)
        plsc.subcore_barrier()
        pltpu.sync_copy(agg.at[pl.ds(s * ZROWS, ZROWS)],
                        out.at[pl.ds(chunk * N_PAD + s * ZROWS, ZROWS)])
        plsc.subcore_barrier()


def _segment_sum(hflat, srcall, dst3, zeros, nchunks):
    npc = nchunks // NC
    kfn = pl.kernel(
        functools.partial(_seg_sum_body, npc),
        out_type=jax.ShapeDtypeStruct((nchunks * N_PAD, LANE), jnp.float32),
        mesh=plsc.VectorSubcoreMesh(core_axis_name="c", subcore_axis_name="s"),
        scratch_types=[
            pltpu.VMEM((NB, LANE), jnp.int32),
            pltpu.VMEM((NB, LANE), jnp.int32),
            pltpu.VMEM((LANE, LANE), jnp.float32),
            pltpu.VMEM_SHARED((N_PAD, LANE), jnp.float32),
            pltpu.SemaphoreType.DMA,
        ],
    )
    return kfn(hflat, srcall, dst3, zeros)


def _init_body(x_ref, w_ref, out_ref):
    xb = x_ref[...]
    h = jnp.dot(xb, w_ref[...], preferred_element_type=jnp.float32)
    msum = jnp.sum(xb[:, :-1], axis=1)
    mask = (msum != 0.0).astype(jnp.float32)[:, None]
    h = h * mask
    for c in range(h.shape[1] // LANE):
        out_ref[c] = h[:, c * LANE:(c + 1) * LANE]


def _init_transform(x, w):
    in_f = x.shape[1]
    d = w.shape[1]
    kc = d // LANE
    return pl.pallas_call(
        _init_body,
        grid=(N_NODES // MB,),
        in_specs=[
            pl.BlockSpec((MB, in_f), lambda i: (i, 0)),
            pl.BlockSpec((in_f, d), lambda i: (0, 0)),
        ],
        out_specs=pl.BlockSpec((kc, MB, LANE), lambda i: (0, i, 0)),
        out_shape=jax.ShapeDtypeStruct((kc, N_NODES, LANE), jnp.float32),
    )(x, w)


def _combine_body(kc, kp, ko, agg_ref, hp_ref, w_ref, b_ref, wres_ref,
                  bres_ref, out_ref):
    conv = b_ref[...]
    for ci in range(kc):
        conv = conv + jnp.dot(agg_ref[ci], w_ref[ci * LANE:(ci + 1) * LANE, :],
                              preferred_element_type=jnp.float32)
    res = bres_ref[...]
    for ci in range(kp):
        res = res + jnp.dot(hp_ref[ci], wres_ref[ci * LANE:(ci + 1) * LANE, :],
                            preferred_element_type=jnp.float32)
    h = jax.nn.relu(conv) + jax.nn.relu(res)
    for co in range(ko):
        out_ref[co] = h[:, co * LANE:(co + 1) * LANE]


def _combine(aggc, hpc, w, b, wres, bres):
    kc = aggc.shape[0]
    kp = hpc.shape[0]
    hid = w.shape[1]
    ko = hid // LANE
    return pl.pallas_call(
        functools.partial(_combine_body, kc, kp, ko),
        grid=(N_NODES // MB,),
        in_specs=[
            pl.BlockSpec((kc, MB, LANE), lambda i: (0, i, 0)),
            pl.BlockSpec((kp, MB, LANE), lambda i: (0, i, 0)),
            pl.BlockSpec(w.shape, lambda i: (0, 0)),
            pl.BlockSpec((1, hid), lambda i: (0, 0)),
            pl.BlockSpec(wres.shape, lambda i: (0, 0)),
            pl.BlockSpec((1, hid), lambda i: (0, 0)),
        ],
        out_specs=pl.BlockSpec((ko, MB, LANE), lambda i: (0, i, 0)),
        out_shape=jax.ShapeDtypeStruct((ko, N_NODES, LANE), jnp.float32),
    )(aggc, hpc, w, b, wres, bres)


def _final_body(kc, kp, agg_ref, hp_ref, w_ref, b_ref, wres_ref, bres_ref,
                x_ref, out_ref):
    conv = b_ref[...]
    for ci in range(kc):
        conv = conv + jnp.dot(agg_ref[ci], w_ref[ci * LANE:(ci + 1) * LANE, :],
                              preferred_element_type=jnp.float32)
    res = bres_ref[...]
    for ci in range(kp):
        res = res + jnp.dot(hp_ref[ci], wres_ref[ci * LANE:(ci + 1) * LANE, :],
                            preferred_element_type=jnp.float32)
    h = jax.nn.relu(conv) + jax.nn.relu(res)
    xb = x_ref[...]
    msum = jnp.sum(xb[:, :-1], axis=1)
    mask = (msum != 0.0).astype(jnp.float32)[:, None]
    out_ref[...] = h * mask


def _final(aggc, hpc, w, b, wres, bres, x):
    kc = aggc.shape[0]
    kp = hpc.shape[0]
    hid = w.shape[1]
    in_f = x.shape[1]
    return pl.pallas_call(
        functools.partial(_final_body, kc, kp),
        grid=(N_NODES // MB,),
        in_specs=[
            pl.BlockSpec((kc, MB, LANE), lambda i: (0, i, 0)),
            pl.BlockSpec((kp, MB, LANE), lambda i: (0, i, 0)),
            pl.BlockSpec(w.shape, lambda i: (0, 0)),
            pl.BlockSpec((1, hid), lambda i: (0, 0)),
            pl.BlockSpec(wres.shape, lambda i: (0, 0)),
            pl.BlockSpec((1, hid), lambda i: (0, 0)),
            pl.BlockSpec((MB, in_f), lambda i: (i, 0)),
        ],
        out_specs=pl.BlockSpec((MB, hid), lambda i: (i, 0)),
        out_shape=jax.ShapeDtypeStruct((N_NODES, hid), jnp.float32),
    )(aggc, hpc, w, b, wres, bres, x)


def kernel(x, edge_index, W_init, W1, b1, Wres1, bres1, W2, b2, Wres2, bres2,
           batch_size):
    src = edge_index[0].astype(jnp.int32)
    dst = edge_index[1].astype(jnp.int32)
    pad = EPAD - N_EDGES
    src_p = jnp.concatenate([src, jnp.zeros((pad,), jnp.int32)])
    dst_p = jnp.concatenate([dst, jnp.full((pad,), N_NODES, jnp.int32)])
    src3 = src_p.reshape(NS, NB, LANE)
    dst3 = dst_p.reshape(NS, NB, LANE)
    offs = (jnp.arange(4, dtype=jnp.int32) * N_NODES)[:, None, None, None]
    src_l1 = (src3[None] + offs).reshape(4 * NS, NB, LANE)
    src_l2 = (src3[None] + offs[:2]).reshape(2 * NS, NB, LANE)
    zeros = jnp.zeros((ZROWS, LANE), jnp.float32)

    h0c = _init_transform(x, W_init)
    agg1 = _segment_sum(h0c.reshape(4 * N_NODES, LANE), src_l1, dst3, zeros, 4)
    h1c = _combine(agg1.reshape(4, N_PAD, LANE), h0c, W1,
                   b1.reshape(1, -1), Wres1, bres1.reshape(1, -1))
    agg2 = _segment_sum(h1c.reshape(2 * N_NODES, LANE), src_l2, dst3, zeros, 2)
    out = _final(agg2.reshape(2, N_PAD, LANE), h1c, W2,
                 b2.reshape(1, -1), Wres2, bres2.reshape(1, -1), x)
    return out.reshape(100, N_NODES // 100, out.shape[-1])


# speedup vs baseline: 3.3478x; 1.2065x over previous
import functools

import jax
import jax.numpy as jnp
from jax import lax
from jax.experimental import pallas as pl
from jax.experimental.pallas import tpu as pltpu
from jax.experimental.pallas import tpu_sc as plsc

N_NODES = 10000
N_EDGES = 160000
LANE = 128
NS = 16
NC = 2
NB = 80
EPS = NB * LANE
EPAD = EPS * NS
N_PAD = 10240
ZROWS = N_PAD // NS
MB = 1000


KBUF = 2
WIN = 16
NW = NB // WIN


def _seg_sum_body(npc, hflat, srcall, dst3, zeros, out, srcw, dstw, rows_v,
                  agg, gsem, isem):
    c = lax.axis_index("c")
    s = lax.axis_index("s")
    for j in range(npc):
        chunk = c * npc + j
        pltpu.sync_copy(zeros, agg.at[pl.ds(s * ZROWS, ZROWS)])
        plsc.subcore_barrier()

        srow = chunk * NS + s
        pltpu.sync_copy(srcall.at[srow, pl.ds(0, WIN)], srcw.at[0])
        pltpu.sync_copy(dst3.at[s, pl.ds(0, WIN)], dstw.at[0])
        pltpu.async_copy(srcall.at[srow, pl.ds(WIN, WIN)], srcw.at[1], isem)
        pltpu.async_copy(dst3.at[s, pl.ds(WIN, WIN)], dstw.at[1], isem)
        for b in range(KBUF):
            pltpu.async_copy(hflat.at[srcw.at[0, b]], rows_v.at[b],
                             gsem.at[b])

        def window(w, carry):
            q = lax.rem(w, 2)
            qn = 1 - q
            for p in range(WIN):
                buf = p % KBUF
                pltpu.make_async_copy(hflat.at[srcw.at[q, p]],
                                      rows_v.at[buf], gsem.at[buf]).wait()
                pltpu.sync_copy(rows_v.at[buf], agg.at[dstw.at[q, p]],
                                add=True)
                if p == WIN - KBUF:
                    @pl.when(w < NW - 1)
                    def _():
                        pltpu.make_async_copy(srcall.at[srow, pl.ds(0, WIN)],
                                              srcw.at[qn], isem).wait()
                        pltpu.make_async_copy(dst3.at[s, pl.ds(0, WIN)],
                                              dstw.at[qn], isem).wait()
                if p < WIN - KBUF:
                    pltpu.async_copy(hflat.at[srcw.at[q, p + KBUF]],
                                     rows_v.at[buf], gsem.at[buf])
                else:
                    @pl.when(w < NW - 1)
                    def _():
                        pltpu.async_copy(
                            hflat.at[srcw.at[qn, p - (WIN - KBUF)]],
                            rows_v.at[buf], gsem.at[buf])

            @pl.when(w < NW - 2)
            def _():
                off = (w + 2) * WIN
                pltpu.async_copy(srcall.at[srow, pl.ds(off, WIN)],
                                 srcw.at[q], isem)
                pltpu.async_copy(dst3.at[s, pl.ds(off, WIN)],
                                 dstw.at[q], isem)

            return carry

        lax.fori_loop(0, NW, window, 0)
        plsc.subcore_barrier()
        pltpu.sync_copy(agg.at[pl.ds(s * ZROWS, ZROWS)],
                        out.at[pl.ds(chunk * N_PAD + s * ZROWS, ZROWS)])
        plsc.subcore_barrier()


def _segment_sum(hflat, srcall, dst3, zeros, nchunks):
    npc = nchunks // NC
    kfn = pl.kernel(
        functools.partial(_seg_sum_body, npc),
        out_type=jax.ShapeDtypeStruct((nchunks * N_PAD, LANE), jnp.float32),
        mesh=plsc.VectorSubcoreMesh(core_axis_name="c", subcore_axis_name="s"),
        scratch_types=[
            pltpu.VMEM((2, WIN, LANE), jnp.int32),
            pltpu.VMEM((2, WIN, LANE), jnp.int32),
            pltpu.VMEM((KBUF, LANE, LANE), jnp.float32),
            pltpu.VMEM_SHARED((N_PAD, LANE), jnp.float32),
            pltpu.SemaphoreType.DMA((KBUF,)),
            pltpu.SemaphoreType.DMA,
        ],
    )
    return kfn(hflat, srcall, dst3, zeros)


def _init_body(x_ref, w_ref, out_ref):
    xb = x_ref[...]
    h = jnp.dot(xb, w_ref[...], preferred_element_type=jnp.float32)
    msum = jnp.sum(xb[:, :-1], axis=1)
    mask = (msum != 0.0).astype(jnp.float32)[:, None]
    h = h * mask
    for c in range(h.shape[1] // LANE):
        out_ref[c] = h[:, c * LANE:(c + 1) * LANE]


def _init_transform(x, w):
    in_f = x.shape[1]
    d = w.shape[1]
    kc = d // LANE
    return pl.pallas_call(
        _init_body,
        grid=(N_NODES // MB,),
        in_specs=[
            pl.BlockSpec((MB, in_f), lambda i: (i, 0)),
            pl.BlockSpec((in_f, d), lambda i: (0, 0)),
        ],
        out_specs=pl.BlockSpec((kc, MB, LANE), lambda i: (0, i, 0)),
        out_shape=jax.ShapeDtypeStruct((kc, N_NODES, LANE), jnp.float32),
    )(x, w)


def _combine_body(kc, kp, ko, agg_ref, hp_ref, w_ref, b_ref, wres_ref,
                  bres_ref, out_ref):
    conv = b_ref[...]
    for ci in range(kc):
        conv = conv + jnp.dot(agg_ref[ci], w_ref[ci * LANE:(ci + 1) * LANE, :],
                              preferred_element_type=jnp.float32)
    res = bres_ref[...]
    for ci in range(kp):
        res = res + jnp.dot(hp_ref[ci], wres_ref[ci * LANE:(ci + 1) * LANE, :],
                            preferred_element_type=jnp.float32)
    h = jax.nn.relu(conv) + jax.nn.relu(res)
    for co in range(ko):
        out_ref[co] = h[:, co * LANE:(co + 1) * LANE]


def _combine(aggc, hpc, w, b, wres, bres):
    kc = aggc.shape[0]
    kp = hpc.shape[0]
    hid = w.shape[1]
    ko = hid // LANE
    return pl.pallas_call(
        functools.partial(_combine_body, kc, kp, ko),
        grid=(N_NODES // MB,),
        in_specs=[
            pl.BlockSpec((kc, MB, LANE), lambda i: (0, i, 0)),
            pl.BlockSpec((kp, MB, LANE), lambda i: (0, i, 0)),
            pl.BlockSpec(w.shape, lambda i: (0, 0)),
            pl.BlockSpec((1, hid), lambda i: (0, 0)),
            pl.BlockSpec(wres.shape, lambda i: (0, 0)),
            pl.BlockSpec((1, hid), lambda i: (0, 0)),
        ],
        out_specs=pl.BlockSpec((ko, MB, LANE), lambda i: (0, i, 0)),
        out_shape=jax.ShapeDtypeStruct((ko, N_NODES, LANE), jnp.float32),
    )(aggc, hpc, w, b, wres, bres)


def _final_body(kc, kp, agg_ref, hp_ref, w_ref, b_ref, wres_ref, bres_ref,
                x_ref, out_ref):
    conv = b_ref[...]
    for ci in range(kc):
        conv = conv + jnp.dot(agg_ref[ci], w_ref[ci * LANE:(ci + 1) * LANE, :],
                              preferred_element_type=jnp.float32)
    res = bres_ref[...]
    for ci in range(kp):
        res = res + jnp.dot(hp_ref[ci], wres_ref[ci * LANE:(ci + 1) * LANE, :],
                            preferred_element_type=jnp.float32)
    h = jax.nn.relu(conv) + jax.nn.relu(res)
    xb = x_ref[...]
    msum = jnp.sum(xb[:, :-1], axis=1)
    mask = (msum != 0.0).astype(jnp.float32)[:, None]
    out_ref[...] = h * mask


def _final(aggc, hpc, w, b, wres, bres, x):
    kc = aggc.shape[0]
    kp = hpc.shape[0]
    hid = w.shape[1]
    in_f = x.shape[1]
    return pl.pallas_call(
        functools.partial(_final_body, kc, kp),
        grid=(N_NODES // MB,),
        in_specs=[
            pl.BlockSpec((kc, MB, LANE), lambda i: (0, i, 0)),
            pl.BlockSpec((kp, MB, LANE), lambda i: (0, i, 0)),
            pl.BlockSpec(w.shape, lambda i: (0, 0)),
            pl.BlockSpec((1, hid), lambda i: (0, 0)),
            pl.BlockSpec(wres.shape, lambda i: (0, 0)),
            pl.BlockSpec((1, hid), lambda i: (0, 0)),
            pl.BlockSpec((MB, in_f), lambda i: (i, 0)),
        ],
        out_specs=pl.BlockSpec((MB, hid), lambda i: (i, 0)),
        out_shape=jax.ShapeDtypeStruct((N_NODES, hid), jnp.float32),
    )(aggc, hpc, w, b, wres, bres, x)


def kernel(x, edge_index, W_init, W1, b1, Wres1, bres1, W2, b2, Wres2, bres2,
           batch_size):
    src = edge_index[0].astype(jnp.int32)
    dst = edge_index[1].astype(jnp.int32)
    pad = EPAD - N_EDGES
    src_p = jnp.concatenate([src, jnp.zeros((pad,), jnp.int32)])
    dst_p = jnp.concatenate([dst, jnp.full((pad,), N_NODES, jnp.int32)])
    src3 = src_p.reshape(NS, NB, LANE)
    dst3 = dst_p.reshape(NS, NB, LANE)
    offs = (jnp.arange(4, dtype=jnp.int32) * N_NODES)[:, None, None, None]
    src_l1 = (src3[None] + offs).reshape(4 * NS, NB, LANE)
    src_l2 = (src3[None] + offs[:2]).reshape(2 * NS, NB, LANE)
    zeros = jnp.zeros((ZROWS, LANE), jnp.float32)

    h0c = _init_transform(x, W_init)
    agg1 = _segment_sum(h0c.reshape(4 * N_NODES, LANE), src_l1, dst3, zeros, 4)
    h1c = _combine(agg1.reshape(4, N_PAD, LANE), h0c, W1,
                   b1.reshape(1, -1), Wres1, bres1.reshape(1, -1))
    agg2 = _segment_sum(h1c.reshape(2 * N_NODES, LANE), src_l2, dst3, zeros, 2)
    out = _final(agg2.reshape(2, N_PAD, LANE), h1c, W2,
                 b2.reshape(1, -1), Wres2, bres2.reshape(1, -1), x)
    return out.reshape(100, N_NODES // 100, out.shape[-1])
